# baseline (speedup 1.0000x reference)
import functools

import jax
import jax.numpy as jnp
import numpy as np
from jax import lax
from jax.experimental import pallas as pl
from jax.experimental.pallas import tpu as pltpu
from jax.experimental.pallas import tpu_sc as plsc

N = 10000
E = 320000
U = 32
DEPTH = 6

NC, NS = 2, 16
NW = NC * NS
EPW = E // NW
CH = 400
NCHUNK = EPW // CH
IR = 100
KSUB = CH // IR
NP = 10240
RPT = NP // NS

NR = N // 4
ER = E // 4
BR = 8000
GRID_E = ER // BR

_F32 = jnp.float32


def _silu(z):
    return z * jax.nn.sigmoid(z)


def _node_init_body(xr, m0, b0, bd1, bd2, bd3, bd4, bt, xo, t1, t2, t3, t4):
    z = jnp.dot(xr[...], m0[...], preferred_element_type=_F32, precision=lax.Precision.HIGHEST) + b0[...]
    xe = _silu(z)
    xo[...] = xe
    t1[...] = jnp.dot(xe, bd1[...], preferred_element_type=_F32, precision=lax.Precision.HIGHEST) + bt[0:1, :]
    t2[...] = jnp.dot(xe, bd2[...], preferred_element_type=_F32, precision=lax.Precision.HIGHEST) + bt[1:2, :]
    t3[...] = jnp.dot(xe, bd3[...], preferred_element_type=_F32, precision=lax.Precision.HIGHEST) + bt[2:3, :]
    t4[...] = jnp.dot(xe, bd4[...], preferred_element_type=_F32, precision=lax.Precision.HIGHEST) + bt[3:4, :]


def _edge_init_body(er, m, b, wo, so):
    z = jnp.dot(er[...], m[...], preferred_element_type=_F32, precision=lax.Precision.HIGHEST) + b[...]
    w = _silu(z)
    wo[...] = w
    so[...] = jax.nn.sigmoid(w)


def _edge_stats_body(w, g, bd, st0, st1):
    h = jnp.dot(w[...], bd[...], preferred_element_type=_F32, precision=lax.Precision.HIGHEST) + g[...]

    @pl.when(pl.program_id(0) == 0)
    def _():
        st0[...] = jnp.zeros_like(st0)
        st1[...] = jnp.zeros_like(st1)

    st0[...] += jnp.sum(h, axis=0, keepdims=True)
    st1[...] += jnp.sum(h * h, axis=0, keepdims=True)


def _edge_update_body(w, g, bd, sc, sh, wo, so):
    h = jnp.dot(w[...], bd[...], preferred_element_type=_F32, precision=lax.Precision.HIGHEST) + g[...]
    z = sc[...] * h + sh[...]
    wn = w[...] + _silu(z)
    wo[...] = wn
    so[...] = jax.nn.sigmoid(wn)


def _edge_update_last_body(w, g, bd, sc, sh, wo):
    h = jnp.dot(w[...], bd[...], preferred_element_type=_F32, precision=lax.Precision.HIGHEST) + g[...]
    z = sc[...] * h + sh[...]
    wo[...] = w[...] + _silu(z)


def _node_update_body(x0, x1, p0, p1, rd, f, ft, g, b,
                      bd1, bd2, bd3, bd4, bt, xo, t1, t2, t3, t4):
    agg = (p0[...] + p1[...]) * rd[...]
    z = x1[...] + agg
    s1 = jnp.dot(jnp.sum(z, axis=0, keepdims=True), f[...],
                 preferred_element_type=_F32, precision=lax.Precision.HIGHEST) * (1.0 / N)
    s2 = jnp.dot(jnp.sum(z * z, axis=0, keepdims=True), f[...],
                 preferred_element_type=_F32, precision=lax.Precision.HIGHEST) * (1.0 / N)
    mu = jnp.dot(s1, ft[...], preferred_element_type=_F32, precision=lax.Precision.HIGHEST)
    var = jnp.maximum(jnp.dot(s2, ft[...], preferred_element_type=_F32, precision=lax.Precision.HIGHEST)
                      - mu * mu, 0.0)
    zn = g[...] * (z - mu) * lax.rsqrt(var + 1e-5) + b[...]
    xn = x0[...] + _silu(zn)
    xo[...] = xn
    t1[...] = jnp.dot(xn, bd1[...], preferred_element_type=_F32, precision=lax.Precision.HIGHEST) + bt[0:1, :]
    t2[...] = jnp.dot(xn, bd2[...], preferred_element_type=_F32, precision=lax.Precision.HIGHEST) + bt[1:2, :]
    t3[...] = jnp.dot(xn, bd3[...], preferred_element_type=_F32, precision=lax.Precision.HIGHEST) + bt[2:3, :]
    t4[...] = jnp.dot(xn, bd4[...], preferred_element_type=_F32, precision=lax.Precision.HIGHEST) + bt[3:4, :]


def _call_node_init(xr, m0, b0, bds, bt):
    outs = [jax.ShapeDtypeStruct((NR, 128), _F32)] * 5
    return pl.pallas_call(
        _node_init_body,
        out_shape=outs,
        interpret=False,
    )(xr, m0, b0, bds[0], bds[1], bds[2], bds[3], bt)


def _call_edge_init(er, m, b):
    grid = (GRID_E,)
    return pl.pallas_call(
        _edge_init_body,
        grid=grid,
        in_specs=[
            pl.BlockSpec((BR, 4), lambda i: (i, 0)),
            pl.BlockSpec((4, 128), lambda i: (0, 0)),
            pl.BlockSpec((1, 128), lambda i: (0, 0)),
        ],
        out_specs=[
            pl.BlockSpec((BR, 128), lambda i: (i, 0)),
            pl.BlockSpec((BR, 128), lambda i: (i, 0)),
        ],
        out_shape=[jax.ShapeDtypeStruct((ER, 128), _F32)] * 2,
        interpret=False,
    )(er, m, b)


def _call_edge_stats(w, g, bd):
    return pl.pallas_call(
        _edge_stats_body,
        grid=(GRID_E,),
        in_specs=[
            pl.BlockSpec((BR, 128), lambda i: (i, 0)),
            pl.BlockSpec((BR, 128), lambda i: (i, 0)),
            pl.BlockSpec((128, 128), lambda i: (0, 0)),
        ],
        out_specs=[
            pl.BlockSpec((1, 128), lambda i: (0, 0)),
            pl.BlockSpec((1, 128), lambda i: (0, 0)),
        ],
        out_shape=[jax.ShapeDtypeStruct((1, 128), _F32)] * 2,
        interpret=False,
    )(w, g, bd)


def _call_edge_update(w, g, bd, sc, sh, last):
    body = _edge_update_last_body if last else _edge_update_body
    n_out = 1 if last else 2
    out = pl.pallas_call(
        body,
        grid=(GRID_E,),
        in_specs=[
            pl.BlockSpec((BR, 128), lambda i: (i, 0)),
            pl.BlockSpec((BR, 128), lambda i: (i, 0)),
            pl.BlockSpec((128, 128), lambda i: (0, 0)),
            pl.BlockSpec((1, 128), lambda i: (0, 0)),
            pl.BlockSpec((1, 128), lambda i: (0, 0)),
        ],
        out_specs=[pl.BlockSpec((BR, 128), lambda i: (i, 0))] * n_out,
        out_shape=[jax.ShapeDtypeStruct((ER, 128), _F32)] * n_out,
        interpret=False,
    )(w, g, bd, sc, sh)
    return out


def _call_node_update(x0, x1, p0, p1, rd, f, ft, g, b, bds, bt):
    outs = [jax.ShapeDtypeStruct((NR, 128), _F32)] * 5
    return pl.pallas_call(
        _node_update_body,
        out_shape=outs,
        interpret=False,
    )(x0, x1, p0, p1, rd, f, ft, g, b, bds[0], bds[1], bds[2], bds[3], bt)


def _sc_layer_body(s_hbm, x2_hbm, x3_hbm, x4_hbm, src_hbm, dst_hbm,
                   accout, gsum, srcb, dstb, sb, ab, bb, pb, acc,
                   sem_i, sem_w, sem_g, sem_s, sem_o):
    cid = lax.axis_index("c")
    sid = lax.axis_index("s")
    wid = sid * NC + cid

    zeros16 = jnp.zeros((16,), _F32)

    def zrow(j, carry):
        pb[0, j, pl.ds(0, 16)] = zeros16
        pb[0, j, pl.ds(16, 16)] = zeros16
        return carry

    lax.fori_loop(0, CH, zrow, 0, unroll=4)
    pltpu.sync_copy(pb.at[0], acc.at[pl.ds(sid * RPT, CH)])
    pltpu.sync_copy(pb.at[0, pl.ds(0, RPT - CH)],
                    acc.at[pl.ds(sid * RPT + CH, RPT - CH)])
    plsc.subcore_barrier()

    def issue_idx_w(c):
        par = c % 2
        p3 = c % 3
        ro = wid * NCHUNK + c
        eo = wid * EPW + c * CH
        return (
            pltpu.async_copy(src_hbm.at[ro], srcb.at[p3], sem_i),
            pltpu.async_copy(dst_hbm.at[ro], dstb.at[p3], sem_i),
            pltpu.async_copy(s_hbm.at[pl.ds(eo, CH)], sb.at[par], sem_w),
        )

    def issue_g1(c):
        par = c % 2
        p3 = c % 3
        cps = []
        for k in range(KSUB):
            dsl = pl.ds(k * IR, IR)
            cps.append(pltpu.async_copy(x2_hbm.at[dstb.at[p3, k]],
                                        ab.at[par, dsl], sem_g))
            cps.append(pltpu.async_copy(x4_hbm.at[dstb.at[p3, k]],
                                        bb.at[par, dsl], sem_g))
        return cps

    def issue_gadd(c):
        par = c % 2
        p3 = c % 3
        return [pltpu.async_copy(x3_hbm.at[srcb.at[p3, k]],
                                 bb.at[par, pl.ds(k * IR, IR)], sem_g,
                                 add=True)
                for k in range(KSUB)]

    d_iw = {0: issue_idx_w(0)}
    d_iw[0][0].wait()
    d_iw[0][1].wait()
    d_g1 = {0: issue_g1(0)}
    d_iw[1] = issue_idx_w(1)
    for cp in d_g1.pop(0):
        cp.wait()
    d_ga = {0: issue_gadd(0)}
    d_iw[1][0].wait()
    d_iw[1][1].wait()
    d_g1[1] = issue_g1(1)
    for cp in d_ga.pop(0):
        cp.wait()
    d_iw[0][2].wait()

    d_sc = {}
    d_out = {}
    for c in range(NCHUNK):
        par = c % 2
        p3 = c % 3
        eo = wid * EPW + c * CH
        if c + 1 < NCHUNK:
            for cp in d_g1.pop(c + 1):
                cp.wait()
            d_ga[c + 1] = issue_gadd(c + 1)

        def mrow(e, carry2):
            lo = pl.ds(0, 16)
            hi = pl.ds(16, 16)
            pb[par, e, lo] = sb[par, e, lo] * ab[par, e, lo]
            pb[par, e, hi] = sb[par, e, hi] * ab[par, e, hi]
            return carry2

        lax.fori_loop(0, CH, mrow, 0, unroll=8)

        d_sc[c] = [pltpu.async_copy(pb.at[par, pl.ds(k * IR, IR)],
                                    acc.at[srcb.at[p3, k]], sem_s, add=True)
                   for k in range(KSUB)]
        d_out[c] = pltpu.async_copy(bb.at[par], gsum.at[pl.ds(eo, CH)],
                                    sem_o)
        if c - 1 >= 0:
            for cp in d_sc.pop(c - 1):
                cp.wait()
        d_out.pop(c).wait()
        if c + 2 < NCHUNK:
            d_iw[c + 2] = issue_idx_w(c + 2)
            d_iw[c + 2][0].wait()
            d_iw[c + 2][1].wait()
            d_g1[c + 2] = issue_g1(c + 2)
        if c + 1 < NCHUNK:
            for cp in d_ga.pop(c + 1):
                cp.wait()
            d_iw.pop(c + 1)[2].wait()

    for cp in d_sc.pop(NCHUNK - 1):
        cp.wait()

    plsc.subcore_barrier()
    pltpu.sync_copy(acc.at[pl.ds(sid * RPT, RPT)],
                    accout.at[cid, pl.ds(sid * RPT, RPT)])


def _sc_deg_body(src_hbm, accout, srcb, onesb, zb, acc):
    cid = lax.axis_index("c")
    sid = lax.axis_index("s")
    wid = sid * NC + cid

    zeros16 = jnp.zeros((16,), _F32)
    ones16 = jnp.ones((16,), _F32)

    def zrow(j, carry):
        zb[j, pl.ds(0, 16)] = zeros16
        zb[j, pl.ds(16, 16)] = zeros16
        return carry

    lax.fori_loop(0, RPT, zrow, 0, unroll=4)

    def orow(j, carry):
        onesb[j, pl.ds(0, 16)] = ones16
        onesb[j, pl.ds(16, 16)] = ones16
        return carry

    lax.fori_loop(0, IR, orow, 0, unroll=4)

    pltpu.sync_copy(zb, acc.at[pl.ds(sid * RPT, RPT)])
    plsc.subcore_barrier()

    def chunk(c, carry):
        ro = wid * NCHUNK + c
        pltpu.sync_copy(src_hbm.at[ro], srcb)
        for k in range(KSUB):
            pltpu.sync_copy(onesb, acc.at[srcb.at[k]], add=True)
        return carry

    lax.fori_loop(0, NCHUNK, chunk, 0)

    plsc.subcore_barrier()
    pltpu.sync_copy(acc.at[pl.ds(sid * RPT, RPT)],
                    accout.at[cid, pl.ds(sid * RPT, RPT)])


_SC_CACHE = {}


def _sc_kernels():
    if not _SC_CACHE:
        mesh = plsc.VectorSubcoreMesh(core_axis_name="c", subcore_axis_name="s",
                                      num_cores=NC, num_subcores=NS)
        _SC_CACHE['layer'] = pl.kernel(
            _sc_layer_body,
            out_type=(jax.ShapeDtypeStruct((NC, NP, U), _F32),
                      jax.ShapeDtypeStruct((E, U), _F32)),
            mesh=mesh,
            scratch_types=[
                pltpu.VMEM((3, KSUB, IR), jnp.int32),
                pltpu.VMEM((3, KSUB, IR), jnp.int32),
                pltpu.VMEM((2, CH, U), _F32),
                pltpu.VMEM((2, CH, U), _F32),
                pltpu.VMEM((2, CH, U), _F32),
                pltpu.VMEM((2, CH, U), _F32),
                pltpu.VMEM_SHARED((NP, U), _F32),
                pltpu.SemaphoreType.DMA,
                pltpu.SemaphoreType.DMA,
                pltpu.SemaphoreType.DMA,
                pltpu.SemaphoreType.DMA,
                pltpu.SemaphoreType.DMA,
            ],
            compiler_params=pltpu.CompilerParams(use_tc_tiling_on_sc=False),
            interpret=False,
        )
        _SC_CACHE['deg'] = pl.kernel(
            _sc_deg_body,
            out_type=jax.ShapeDtypeStruct((NC, NP, U), _F32),
            mesh=mesh,
            scratch_types=[
                pltpu.VMEM((KSUB, IR), jnp.int32),
                pltpu.VMEM((IR, U), _F32),
                pltpu.VMEM((RPT, U), _F32),
                pltpu.VMEM_SHARED((NP, U), _F32),
            ],
            compiler_params=pltpu.CompilerParams(use_tc_tiling_on_sc=False),
            interpret=False,
        )
    return _SC_CACHE


def _run_sc_layer(s, x2t, x3t, x4t, src2, dst2):
    return _sc_kernels()['layer'](s, x2t, x3t, x4t, src2, dst2)


def _run_sc_deg(src2):
    return _sc_kernels()['deg'](src2)


def _bd(w):
    return jnp.kron(jnp.eye(4, dtype=_F32), w.T)


def _tile128(v):
    return jnp.tile(v.reshape(1, U), (1, 4)).reshape(1, 128)


def _branch(feat, edge_attr, src2, dst2, rd, p):
    feats = feat.shape[1]
    xr = feat.reshape(NR, 4 * feats)
    m0 = jnp.kron(jnp.eye(4, dtype=_F32), p['v_lin0_W'].T)
    b0 = _tile128(p['v_lin0_b'])

    fold = jnp.tile(jnp.eye(U, dtype=_F32), (4, 1))
    foldt = fold.T

    def layer_mats(i):
        bds = [_bd(p['v1_W'][i]), _bd(p['v2_W'][i]),
               _bd(p['v3_W'][i]), _bd(p['v4_W'][i])]
        bt = jnp.stack([
            _tile128(p['v1_b'][i])[0],
            _tile128(p['v2_b'][i])[0],
            _tile128(p['v3_b'][i])[0],
            _tile128(p['v4_b'][i] + p['e0_b'][i])[0],
        ], axis=0)
        return bds, bt

    bds0, bt0 = layer_mats(0)
    x0, x1t, x2t, x3t, x4t = _call_node_init(xr, m0, b0, bds0, bt0)

    er = edge_attr.reshape(ER, 4)
    me = jnp.kron(jnp.eye(4, dtype=_F32), p['e_lin0_W'].T)
    be = _tile128(p['e_lin0_b'])
    w, s = _call_edge_init(er, me, be)

    for i in range(DEPTH):
        x2flat = x2t.reshape(N, U)
        x3flat = x3t.reshape(N, U)
        x4flat = x4t.reshape(N, U)
        sflat = s.reshape(E, U)
        accp, gsum = _run_sc_layer(sflat, x2flat, x3flat, x4flat, src2, dst2)
        g128 = gsum.reshape(ER, 128)

        bde = _bd(p['e0_W'][i])
        st0, st1 = _call_edge_stats(w, g128, bde)
        s0 = jnp.sum(st0.reshape(4, U), axis=0)
        s1 = jnp.sum(st1.reshape(4, U), axis=0)
        mu = s0 / E
        var = jnp.maximum(s1 / E - mu * mu, 0.0)
        scale = p['e_bn_g'][i] * lax.rsqrt(var + 1e-5)
        shift = p['e_bn_b'][i] - mu * scale
        sc128 = _tile128(scale)
        sh128 = _tile128(shift)

        last = (i == DEPTH - 1)
        if last:
            (w,) = _call_edge_update(w, g128, bde, sc128, sh128, True)
        else:
            w, s = _call_edge_update(w, g128, bde, sc128, sh128, False)

            p0 = accp[0, :N].reshape(NR, 128)
            p1 = accp[1, :N].reshape(NR, 128)
            gv = _tile128(p['v_bn_g'][i])
            bv = _tile128(p['v_bn_b'][i])
            bdsn, btn = layer_mats(i + 1)
            x0, x1t, x2t, x3t, x4t = _call_node_update(
                x0, x1t, p0, p1, rd, fold, foldt, gv, bv, bdsn, btn)

    return w.reshape(E, U)


def kernel(x, y, edge_index, edge_attr, params_cvrp, params_tw):
    src = edge_index[0]
    dst = edge_index[1]
    src2 = src.reshape(NW * NCHUNK, KSUB, IR)
    dst2 = dst.reshape(NW * NCHUNK, KSUB, IR)

    degp = _run_sc_deg(src2)
    deg = degp[0, :N, 0] + degp[1, :N, 0]
    rdeg = 1.0 / jnp.clip(deg, 1.0)
    rd = jnp.broadcast_to(rdeg[:, None], (N, U)).reshape(NR, 128)

    w_cvrp = _branch(x, edge_attr, src2, dst2, rd, params_cvrp)
    w_tw = _branch(y, edge_attr, src2, dst2, rd, params_tw)
    return (w_cvrp, w_tw)


# speedup vs baseline: 1.0267x; 1.0267x over previous
import functools

import jax
import jax.numpy as jnp
import numpy as np
from jax import lax
from jax.experimental import pallas as pl
from jax.experimental.pallas import tpu as pltpu
from jax.experimental.pallas import tpu_sc as plsc

N = 10000
E = 320000
U = 32
DEPTH = 6

NC, NS = 2, 16
NW = NC * NS
EPW = E // NW
CH = 400
NCHUNK = EPW // CH
IR = 100
KSUB = CH // IR
NP = 10240
RPT = NP // NS

NR = N // 4
ER = E // 4
BR = 8000
GRID_E = ER // BR

_F32 = jnp.float32


def _silu(z):
    return z * jax.nn.sigmoid(z)


def _node_init_body(xr, m0, b0, bd1, bd2, bd3, bd4, bt, xo, t1, t2, t3, t4):
    z = jnp.dot(xr[...], m0[...], preferred_element_type=_F32, precision=lax.Precision.HIGHEST) + b0[...]
    xe = _silu(z)
    xo[...] = xe
    t1[...] = jnp.dot(xe, bd1[...], preferred_element_type=_F32, precision=lax.Precision.HIGHEST) + bt[0:1, :]
    t2[...] = jnp.dot(xe, bd2[...], preferred_element_type=_F32, precision=lax.Precision.HIGHEST) + bt[1:2, :]
    t3[...] = jnp.dot(xe, bd3[...], preferred_element_type=_F32, precision=lax.Precision.HIGHEST) + bt[2:3, :]
    t4[...] = jnp.dot(xe, bd4[...], preferred_element_type=_F32, precision=lax.Precision.HIGHEST) + bt[3:4, :]


def _edge_init_body(er, m, b, wo, so):
    z = jnp.dot(er[...], m[...], preferred_element_type=_F32, precision=lax.Precision.HIGHEST) + b[...]
    w = _silu(z)
    wo[...] = w
    so[...] = jax.nn.sigmoid(w)


def _edge_stats_body(w, g, bd, st0, st1):
    h = jnp.dot(w[...], bd[...], preferred_element_type=_F32, precision=lax.Precision.HIGHEST) + g[...]

    @pl.when(pl.program_id(0) == 0)
    def _():
        st0[...] = jnp.zeros_like(st0)
        st1[...] = jnp.zeros_like(st1)

    st0[...] += jnp.sum(h, axis=0, keepdims=True)
    st1[...] += jnp.sum(h * h, axis=0, keepdims=True)


def _edge_update_body(w, g, bd, sc, sh, wo, so):
    h = jnp.dot(w[...], bd[...], preferred_element_type=_F32, precision=lax.Precision.HIGHEST) + g[...]
    z = sc[...] * h + sh[...]
    wn = w[...] + _silu(z)
    wo[...] = wn
    so[...] = jax.nn.sigmoid(wn)


def _edge_update_last_body(w, g, bd, sc, sh, wo):
    h = jnp.dot(w[...], bd[...], preferred_element_type=_F32, precision=lax.Precision.HIGHEST) + g[...]
    z = sc[...] * h + sh[...]
    wo[...] = w[...] + _silu(z)


def _node_update_body(x0, x1, p0, p1, rd, f, ft, g, b,
                      bd1, bd2, bd3, bd4, bt, xo, t1, t2, t3, t4):
    agg = (p0[...] + p1[...]) * rd[...]
    z = x1[...] + agg
    s1 = jnp.dot(jnp.sum(z, axis=0, keepdims=True), f[...],
                 preferred_element_type=_F32, precision=lax.Precision.HIGHEST) * (1.0 / N)
    s2 = jnp.dot(jnp.sum(z * z, axis=0, keepdims=True), f[...],
                 preferred_element_type=_F32, precision=lax.Precision.HIGHEST) * (1.0 / N)
    mu = jnp.dot(s1, ft[...], preferred_element_type=_F32, precision=lax.Precision.HIGHEST)
    var = jnp.maximum(jnp.dot(s2, ft[...], preferred_element_type=_F32, precision=lax.Precision.HIGHEST)
                      - mu * mu, 0.0)
    zn = g[...] * (z - mu) * lax.rsqrt(var + 1e-5) + b[...]
    xn = x0[...] + _silu(zn)
    xo[...] = xn
    t1[...] = jnp.dot(xn, bd1[...], preferred_element_type=_F32, precision=lax.Precision.HIGHEST) + bt[0:1, :]
    t2[...] = jnp.dot(xn, bd2[...], preferred_element_type=_F32, precision=lax.Precision.HIGHEST) + bt[1:2, :]
    t3[...] = jnp.dot(xn, bd3[...], preferred_element_type=_F32, precision=lax.Precision.HIGHEST) + bt[2:3, :]
    t4[...] = jnp.dot(xn, bd4[...], preferred_element_type=_F32, precision=lax.Precision.HIGHEST) + bt[3:4, :]


def _call_node_init(xr, m0, b0, bds, bt):
    outs = [jax.ShapeDtypeStruct((NR, 128), _F32)] * 5
    return pl.pallas_call(
        _node_init_body,
        out_shape=outs,
        interpret=False,
    )(xr, m0, b0, bds[0], bds[1], bds[2], bds[3], bt)


def _call_edge_init(er, m, b):
    grid = (GRID_E,)
    return pl.pallas_call(
        _edge_init_body,
        grid=grid,
        in_specs=[
            pl.BlockSpec((BR, 4), lambda i: (i, 0)),
            pl.BlockSpec((4, 128), lambda i: (0, 0)),
            pl.BlockSpec((1, 128), lambda i: (0, 0)),
        ],
        out_specs=[
            pl.BlockSpec((BR, 128), lambda i: (i, 0)),
            pl.BlockSpec((BR, 128), lambda i: (i, 0)),
        ],
        out_shape=[jax.ShapeDtypeStruct((ER, 128), _F32)] * 2,
        interpret=False,
    )(er, m, b)


def _call_edge_stats(w, g, bd):
    return pl.pallas_call(
        _edge_stats_body,
        grid=(GRID_E,),
        in_specs=[
            pl.BlockSpec((BR, 128), lambda i: (i, 0)),
            pl.BlockSpec((BR, 128), lambda i: (i, 0)),
            pl.BlockSpec((128, 128), lambda i: (0, 0)),
        ],
        out_specs=[
            pl.BlockSpec((1, 128), lambda i: (0, 0)),
            pl.BlockSpec((1, 128), lambda i: (0, 0)),
        ],
        out_shape=[jax.ShapeDtypeStruct((1, 128), _F32)] * 2,
        interpret=False,
    )(w, g, bd)


def _call_edge_update(w, g, bd, sc, sh, last):
    body = _edge_update_last_body if last else _edge_update_body
    n_out = 1 if last else 2
    out = pl.pallas_call(
        body,
        grid=(GRID_E,),
        in_specs=[
            pl.BlockSpec((BR, 128), lambda i: (i, 0)),
            pl.BlockSpec((BR, 128), lambda i: (i, 0)),
            pl.BlockSpec((128, 128), lambda i: (0, 0)),
            pl.BlockSpec((1, 128), lambda i: (0, 0)),
            pl.BlockSpec((1, 128), lambda i: (0, 0)),
        ],
        out_specs=[pl.BlockSpec((BR, 128), lambda i: (i, 0))] * n_out,
        out_shape=[jax.ShapeDtypeStruct((ER, 128), _F32)] * n_out,
        interpret=False,
    )(w, g, bd, sc, sh)
    return out


def _call_node_update(x0, x1, p0, p1, rd, f, ft, g, b, bds, bt):
    outs = [jax.ShapeDtypeStruct((NR, 128), _F32)] * 5
    return pl.pallas_call(
        _node_update_body,
        out_shape=outs,
        interpret=False,
    )(x0, x1, p0, p1, rd, f, ft, g, b, bds[0], bds[1], bds[2], bds[3], bt)


def _sc_layer_body(s_hbm, x2_hbm, x3_hbm, x4_hbm, src_hbm, dst_hbm,
                   accout, gsum, srcb, dstb, sb, ab, bb, pb, acc,
                   sem_i, sem_w, sem_g, sem_s, sem_o):
    cid = lax.axis_index("c")
    sid = lax.axis_index("s")
    wid = sid * NC + cid

    zeros16 = jnp.zeros((16,), _F32)

    def zrow(j, carry):
        pb[0, j, pl.ds(0, 16)] = zeros16
        pb[0, j, pl.ds(16, 16)] = zeros16
        return carry

    lax.fori_loop(0, CH, zrow, 0, unroll=4)
    pltpu.sync_copy(pb.at[0], acc.at[pl.ds(sid * RPT, CH)])
    pltpu.sync_copy(pb.at[0, pl.ds(0, RPT - CH)],
                    acc.at[pl.ds(sid * RPT + CH, RPT - CH)])
    plsc.subcore_barrier()

    def issue_idx_w(c):
        par = c % 2
        p3 = c % 3
        ro = wid * NCHUNK + c
        eo = wid * EPW + c * CH
        return (
            pltpu.async_copy(src_hbm.at[ro], srcb.at[p3], sem_i),
            pltpu.async_copy(dst_hbm.at[ro], dstb.at[p3], sem_i),
            pltpu.async_copy(s_hbm.at[pl.ds(eo, CH)], sb.at[par], sem_w),
        )

    def issue_g1(c):
        par = c % 2
        p3 = c % 3
        cps = []
        for k in range(KSUB):
            dsl = pl.ds(k * IR, IR)
            cps.append(pltpu.async_copy(x2_hbm.at[dstb.at[p3, k]],
                                        ab.at[par, dsl], sem_g))
            cps.append(pltpu.async_copy(x4_hbm.at[dstb.at[p3, k]],
                                        bb.at[par, dsl], sem_g))
        return cps

    def issue_gadd(c):
        par = c % 2
        p3 = c % 3
        return [pltpu.async_copy(x3_hbm.at[srcb.at[p3, k]],
                                 bb.at[par, pl.ds(k * IR, IR)], sem_g,
                                 add=True)
                for k in range(KSUB)]

    d_iw = {0: issue_idx_w(0)}
    d_iw[0][0].wait()
    d_iw[0][1].wait()
    d_g1 = {0: issue_g1(0)}
    d_iw[1] = issue_idx_w(1)
    for cp in d_g1.pop(0):
        cp.wait()
    d_ga = {0: issue_gadd(0)}
    d_iw[1][0].wait()
    d_iw[1][1].wait()
    d_g1[1] = issue_g1(1)
    for cp in d_ga.pop(0):
        cp.wait()
    d_iw[0][2].wait()

    d_sc = {}
    d_out = {}
    for c in range(NCHUNK):
        par = c % 2
        p3 = c % 3
        eo = wid * EPW + c * CH
        if c + 1 < NCHUNK:
            for cp in d_g1.pop(c + 1):
                cp.wait()
            d_ga[c + 1] = issue_gadd(c + 1)

        def mrow(e, carry2):
            lo = pl.ds(0, 16)
            hi = pl.ds(16, 16)
            pb[par, e, lo] = sb[par, e, lo] * ab[par, e, lo]
            pb[par, e, hi] = sb[par, e, hi] * ab[par, e, hi]
            return carry2

        lax.fori_loop(0, CH, mrow, 0, unroll=8)

        d_sc[c] = [pltpu.async_copy(pb.at[par, pl.ds(k * IR, IR)],
                                    acc.at[srcb.at[p3, k]], sem_s, add=True)
                   for k in range(KSUB)]
        d_out[c] = pltpu.async_copy(bb.at[par], gsum.at[pl.ds(eo, CH)],
                                    sem_o)
        if c - 1 >= 0:
            for cp in d_sc.pop(c - 1):
                cp.wait()
        if c + 2 < NCHUNK:
            d_iw[c + 2] = issue_idx_w(c + 2)
            d_iw[c + 2][0].wait()
            d_iw[c + 2][1].wait()
            d_out.pop(c).wait()
            d_g1[c + 2] = issue_g1(c + 2)
        else:
            d_out.pop(c).wait()
        if c + 1 < NCHUNK:
            for cp in d_ga.pop(c + 1):
                cp.wait()
            d_iw.pop(c + 1)[2].wait()

    for cp in d_sc.pop(NCHUNK - 1):
        cp.wait()

    plsc.subcore_barrier()
    pltpu.sync_copy(acc.at[pl.ds(sid * RPT, RPT)],
                    accout.at[cid, pl.ds(sid * RPT, RPT)])


def _sc_deg_body(src_hbm, accout, srcb, onesb, zb, acc):
    cid = lax.axis_index("c")
    sid = lax.axis_index("s")
    wid = sid * NC + cid

    zeros16 = jnp.zeros((16,), _F32)
    ones16 = jnp.ones((16,), _F32)

    def zrow(j, carry):
        zb[j, pl.ds(0, 16)] = zeros16
        zb[j, pl.ds(16, 16)] = zeros16
        return carry

    lax.fori_loop(0, RPT, zrow, 0, unroll=4)

    def orow(j, carry):
        onesb[j, pl.ds(0, 16)] = ones16
        onesb[j, pl.ds(16, 16)] = ones16
        return carry

    lax.fori_loop(0, IR, orow, 0, unroll=4)

    pltpu.sync_copy(zb, acc.at[pl.ds(sid * RPT, RPT)])
    plsc.subcore_barrier()

    def chunk(c, carry):
        ro = wid * NCHUNK + c
        pltpu.sync_copy(src_hbm.at[ro], srcb)
        for k in range(KSUB):
            pltpu.sync_copy(onesb, acc.at[srcb.at[k]], add=True)
        return carry

    lax.fori_loop(0, NCHUNK, chunk, 0)

    plsc.subcore_barrier()
    pltpu.sync_copy(acc.at[pl.ds(sid * RPT, RPT)],
                    accout.at[cid, pl.ds(sid * RPT, RPT)])


_SC_CACHE = {}


def _sc_kernels():
    if not _SC_CACHE:
        mesh = plsc.VectorSubcoreMesh(core_axis_name="c", subcore_axis_name="s",
                                      num_cores=NC, num_subcores=NS)
        _SC_CACHE['layer'] = pl.kernel(
            _sc_layer_body,
            out_type=(jax.ShapeDtypeStruct((NC, NP, U), _F32),
                      jax.ShapeDtypeStruct((E, U), _F32)),
            mesh=mesh,
            scratch_types=[
                pltpu.VMEM((3, KSUB, IR), jnp.int32),
                pltpu.VMEM((3, KSUB, IR), jnp.int32),
                pltpu.VMEM((2, CH, U), _F32),
                pltpu.VMEM((2, CH, U), _F32),
                pltpu.VMEM((2, CH, U), _F32),
                pltpu.VMEM((2, CH, U), _F32),
                pltpu.VMEM_SHARED((NP, U), _F32),
                pltpu.SemaphoreType.DMA,
                pltpu.SemaphoreType.DMA,
                pltpu.SemaphoreType.DMA,
                pltpu.SemaphoreType.DMA,
                pltpu.SemaphoreType.DMA,
            ],
            compiler_params=pltpu.CompilerParams(use_tc_tiling_on_sc=False),
            interpret=False,
        )
        _SC_CACHE['deg'] = pl.kernel(
            _sc_deg_body,
            out_type=jax.ShapeDtypeStruct((NC, NP, U), _F32),
            mesh=mesh,
            scratch_types=[
                pltpu.VMEM((KSUB, IR), jnp.int32),
                pltpu.VMEM((IR, U), _F32),
                pltpu.VMEM((RPT, U), _F32),
                pltpu.VMEM_SHARED((NP, U), _F32),
            ],
            compiler_params=pltpu.CompilerParams(use_tc_tiling_on_sc=False),
            interpret=False,
        )
    return _SC_CACHE


def _run_sc_layer(s, x2t, x3t, x4t, src2, dst2):
    return _sc_kernels()['layer'](s, x2t, x3t, x4t, src2, dst2)


def _run_sc_deg(src2):
    return _sc_kernels()['deg'](src2)


def _bd(w):
    return jnp.kron(jnp.eye(4, dtype=_F32), w.T)


def _tile128(v):
    return jnp.tile(v.reshape(1, U), (1, 4)).reshape(1, 128)


def _branch(feat, edge_attr, src2, dst2, rd, p):
    feats = feat.shape[1]
    xr = feat.reshape(NR, 4 * feats)
    m0 = jnp.kron(jnp.eye(4, dtype=_F32), p['v_lin0_W'].T)
    b0 = _tile128(p['v_lin0_b'])

    fold = jnp.tile(jnp.eye(U, dtype=_F32), (4, 1))
    foldt = fold.T

    def layer_mats(i):
        bds = [_bd(p['v1_W'][i]), _bd(p['v2_W'][i]),
               _bd(p['v3_W'][i]), _bd(p['v4_W'][i])]
        bt = jnp.stack([
            _tile128(p['v1_b'][i])[0],
            _tile128(p['v2_b'][i])[0],
            _tile128(p['v3_b'][i])[0],
            _tile128(p['v4_b'][i] + p['e0_b'][i])[0],
        ], axis=0)
        return bds, bt

    bds0, bt0 = layer_mats(0)
    x0, x1t, x2t, x3t, x4t = _call_node_init(xr, m0, b0, bds0, bt0)

    er = edge_attr.reshape(ER, 4)
    me = jnp.kron(jnp.eye(4, dtype=_F32), p['e_lin0_W'].T)
    be = _tile128(p['e_lin0_b'])
    w, s = _call_edge_init(er, me, be)

    for i in range(DEPTH):
        x2flat = x2t.reshape(N, U)
        x3flat = x3t.reshape(N, U)
        x4flat = x4t.reshape(N, U)
        sflat = s.reshape(E, U)
        accp, gsum = _run_sc_layer(sflat, x2flat, x3flat, x4flat, src2, dst2)
        g128 = gsum.reshape(ER, 128)

        bde = _bd(p['e0_W'][i])
        st0, st1 = _call_edge_stats(w, g128, bde)
        s0 = jnp.sum(st0.reshape(4, U), axis=0)
        s1 = jnp.sum(st1.reshape(4, U), axis=0)
        mu = s0 / E
        var = jnp.maximum(s1 / E - mu * mu, 0.0)
        scale = p['e_bn_g'][i] * lax.rsqrt(var + 1e-5)
        shift = p['e_bn_b'][i] - mu * scale
        sc128 = _tile128(scale)
        sh128 = _tile128(shift)

        last = (i == DEPTH - 1)
        if last:
            (w,) = _call_edge_update(w, g128, bde, sc128, sh128, True)
        else:
            w, s = _call_edge_update(w, g128, bde, sc128, sh128, False)

            p0 = accp[0, :N].reshape(NR, 128)
            p1 = accp[1, :N].reshape(NR, 128)
            gv = _tile128(p['v_bn_g'][i])
            bv = _tile128(p['v_bn_b'][i])
            bdsn, btn = layer_mats(i + 1)
            x0, x1t, x2t, x3t, x4t = _call_node_update(
                x0, x1t, p0, p1, rd, fold, foldt, gv, bv, bdsn, btn)

    return w.reshape(E, U)


def kernel(x, y, edge_index, edge_attr, params_cvrp, params_tw):
    src = edge_index[0]
    dst = edge_index[1]
    src2 = src.reshape(NW * NCHUNK, KSUB, IR)
    dst2 = dst.reshape(NW * NCHUNK, KSUB, IR)

    degp = _run_sc_deg(src2)
    deg = degp[0, :N, 0] + degp[1, :N, 0]
    rdeg = 1.0 / jnp.clip(deg, 1.0)
    rd = jnp.broadcast_to(rdeg[:, None], (N, U)).reshape(NR, 128)

    w_cvrp = _branch(x, edge_attr, src2, dst2, rd, params_cvrp)
    w_tw = _branch(y, edge_attr, src2, dst2, rd, params_tw)
    return (w_cvrp, w_tw)


# speedup vs baseline: 1.1169x; 1.0878x over previous
import functools

import jax
import jax.numpy as jnp
import numpy as np
from jax import lax
from jax.experimental import pallas as pl
from jax.experimental.pallas import tpu as pltpu
from jax.experimental.pallas import tpu_sc as plsc

N = 10000
E = 320000
U = 32
DEPTH = 6

NC, NS = 2, 16
NW = NC * NS
EPW = E // NW
CH = 200
NCHUNK = EPW // CH
IR = 100
KSUB = CH // IR
NP = 10240
RPT = NP // NS

NR = N // 4
ER = E // 4
BR = 8000
GRID_E = ER // BR

_F32 = jnp.float32


def _silu(z):
    return z * jax.nn.sigmoid(z)


def _node_init_body(xr, m0, b0, bd1, bd2, bd3, bd4, bt, xo, t1, t2, t3, t4):
    z = jnp.dot(xr[...], m0[...], preferred_element_type=_F32, precision=lax.Precision.HIGHEST) + b0[...]
    xe = _silu(z)
    xo[...] = xe
    t1[...] = jnp.dot(xe, bd1[...], preferred_element_type=_F32, precision=lax.Precision.HIGHEST) + bt[0:1, :]
    t2[...] = jnp.dot(xe, bd2[...], preferred_element_type=_F32, precision=lax.Precision.HIGHEST) + bt[1:2, :]
    t3[...] = jnp.dot(xe, bd3[...], preferred_element_type=_F32, precision=lax.Precision.HIGHEST) + bt[2:3, :]
    t4[...] = jnp.dot(xe, bd4[...], preferred_element_type=_F32, precision=lax.Precision.HIGHEST) + bt[3:4, :]


def _edge_init_body(er, m, b, wo, so):
    z = jnp.dot(er[...], m[...], preferred_element_type=_F32, precision=lax.Precision.HIGHEST) + b[...]
    w = _silu(z)
    wo[...] = w
    so[...] = jax.nn.sigmoid(w)


def _edge_stats_body(w, g, bd, st0, st1):
    h = jnp.dot(w[...], bd[...], preferred_element_type=_F32, precision=lax.Precision.HIGHEST) + g[...]

    @pl.when(pl.program_id(0) == 0)
    def _():
        st0[...] = jnp.zeros_like(st0)
        st1[...] = jnp.zeros_like(st1)

    st0[...] += jnp.sum(h, axis=0, keepdims=True)
    st1[...] += jnp.sum(h * h, axis=0, keepdims=True)


def _edge_update_body(w, g, bd, sc, sh, wo, so):
    h = jnp.dot(w[...], bd[...], preferred_element_type=_F32, precision=lax.Precision.HIGHEST) + g[...]
    z = sc[...] * h + sh[...]
    wn = w[...] + _silu(z)
    wo[...] = wn
    so[...] = jax.nn.sigmoid(wn)


def _edge_update_last_body(w, g, bd, sc, sh, wo):
    h = jnp.dot(w[...], bd[...], preferred_element_type=_F32, precision=lax.Precision.HIGHEST) + g[...]
    z = sc[...] * h + sh[...]
    wo[...] = w[...] + _silu(z)


def _node_update_body(x0, x1, p0, p1, rd, f, ft, g, b,
                      bd1, bd2, bd3, bd4, bt, xo, t1, t2, t3, t4):
    agg = (p0[...] + p1[...]) * rd[...]
    z = x1[...] + agg
    s1 = jnp.dot(jnp.sum(z, axis=0, keepdims=True), f[...],
                 preferred_element_type=_F32, precision=lax.Precision.HIGHEST) * (1.0 / N)
    s2 = jnp.dot(jnp.sum(z * z, axis=0, keepdims=True), f[...],
                 preferred_element_type=_F32, precision=lax.Precision.HIGHEST) * (1.0 / N)
    mu = jnp.dot(s1, ft[...], preferred_element_type=_F32, precision=lax.Precision.HIGHEST)
    var = jnp.maximum(jnp.dot(s2, ft[...], preferred_element_type=_F32, precision=lax.Precision.HIGHEST)
                      - mu * mu, 0.0)
    zn = g[...] * (z - mu) * lax.rsqrt(var + 1e-5) + b[...]
    xn = x0[...] + _silu(zn)
    xo[...] = xn
    t1[...] = jnp.dot(xn, bd1[...], preferred_element_type=_F32, precision=lax.Precision.HIGHEST) + bt[0:1, :]
    t2[...] = jnp.dot(xn, bd2[...], preferred_element_type=_F32, precision=lax.Precision.HIGHEST) + bt[1:2, :]
    t3[...] = jnp.dot(xn, bd3[...], preferred_element_type=_F32, precision=lax.Precision.HIGHEST) + bt[2:3, :]
    t4[...] = jnp.dot(xn, bd4[...], preferred_element_type=_F32, precision=lax.Precision.HIGHEST) + bt[3:4, :]


def _call_node_init(xr, m0, b0, bds, bt):
    outs = [jax.ShapeDtypeStruct((NR, 128), _F32)] * 5
    return pl.pallas_call(
        _node_init_body,
        out_shape=outs,
        interpret=False,
    )(xr, m0, b0, bds[0], bds[1], bds[2], bds[3], bt)


def _call_edge_init(er, m, b):
    grid = (GRID_E,)
    return pl.pallas_call(
        _edge_init_body,
        grid=grid,
        in_specs=[
            pl.BlockSpec((BR, 4), lambda i: (i, 0)),
            pl.BlockSpec((4, 128), lambda i: (0, 0)),
            pl.BlockSpec((1, 128), lambda i: (0, 0)),
        ],
        out_specs=[
            pl.BlockSpec((BR, 128), lambda i: (i, 0)),
            pl.BlockSpec((BR, 128), lambda i: (i, 0)),
        ],
        out_shape=[jax.ShapeDtypeStruct((ER, 128), _F32)] * 2,
        interpret=False,
    )(er, m, b)


def _call_edge_stats(w, g, bd):
    return pl.pallas_call(
        _edge_stats_body,
        grid=(GRID_E,),
        in_specs=[
            pl.BlockSpec((BR, 128), lambda i: (i, 0)),
            pl.BlockSpec((BR, 128), lambda i: (i, 0)),
            pl.BlockSpec((128, 128), lambda i: (0, 0)),
        ],
        out_specs=[
            pl.BlockSpec((1, 128), lambda i: (0, 0)),
            pl.BlockSpec((1, 128), lambda i: (0, 0)),
        ],
        out_shape=[jax.ShapeDtypeStruct((1, 128), _F32)] * 2,
        interpret=False,
    )(w, g, bd)


def _call_edge_update(w, g, bd, sc, sh, last):
    body = _edge_update_last_body if last else _edge_update_body
    n_out = 1 if last else 2
    out = pl.pallas_call(
        body,
        grid=(GRID_E,),
        in_specs=[
            pl.BlockSpec((BR, 128), lambda i: (i, 0)),
            pl.BlockSpec((BR, 128), lambda i: (i, 0)),
            pl.BlockSpec((128, 128), lambda i: (0, 0)),
            pl.BlockSpec((1, 128), lambda i: (0, 0)),
            pl.BlockSpec((1, 128), lambda i: (0, 0)),
        ],
        out_specs=[pl.BlockSpec((BR, 128), lambda i: (i, 0))] * n_out,
        out_shape=[jax.ShapeDtypeStruct((ER, 128), _F32)] * n_out,
        interpret=False,
    )(w, g, bd, sc, sh)
    return out


def _call_node_update(x0, x1, p0, p1, rd, f, ft, g, b, bds, bt):
    outs = [jax.ShapeDtypeStruct((NR, 128), _F32)] * 5
    return pl.pallas_call(
        _node_update_body,
        out_shape=outs,
        interpret=False,
    )(x0, x1, p0, p1, rd, f, ft, g, b, bds[0], bds[1], bds[2], bds[3], bt)


def _sc_layer_body(s_hbm, x2_hbm, x3_hbm, x4_hbm, src_hbm, dst_hbm,
                   accout, gsum, srcb, dstb, sb, ab, bb, pb, acc,
                   sem_i, sem_w, sem_g, sem_s, sem_o):
    cid = lax.axis_index("c")
    sid = lax.axis_index("s")
    wid = sid * NC + cid

    zeros16 = jnp.zeros((16,), _F32)

    def zrow(j, carry):
        pb[0, j, pl.ds(0, 16)] = zeros16
        pb[0, j, pl.ds(16, 16)] = zeros16
        pb[1, j, pl.ds(0, 16)] = zeros16
        pb[1, j, pl.ds(16, 16)] = zeros16
        return carry

    lax.fori_loop(0, CH, zrow, 0, unroll=4)
    pltpu.sync_copy(pb.at[0], acc.at[pl.ds(sid * RPT, CH)])
    pltpu.sync_copy(pb.at[1], acc.at[pl.ds(sid * RPT + CH, CH)])
    pltpu.sync_copy(pb.at[0], acc.at[pl.ds(sid * RPT + 2 * CH, CH)])
    pltpu.sync_copy(pb.at[1, pl.ds(0, RPT - 3 * CH)],
                    acc.at[pl.ds(sid * RPT + 3 * CH, RPT - 3 * CH)])
    plsc.subcore_barrier()

    def issue_idx_w(c):
        p3 = c % 3
        p4 = c % 4
        ro = wid * NCHUNK + c
        eo = wid * EPW + c * CH
        return (
            pltpu.async_copy(src_hbm.at[ro], srcb.at[p4], sem_i),
            pltpu.async_copy(dst_hbm.at[ro], dstb.at[p4], sem_i),
            pltpu.async_copy(s_hbm.at[pl.ds(eo, CH)], sb.at[p3], sem_w),
        )

    def issue_g1(c):
        p3 = c % 3
        p4 = c % 4
        cps = []
        for k in range(KSUB):
            dsl = pl.ds(k * IR, IR)
            cps.append(pltpu.async_copy(x2_hbm.at[dstb.at[p4, k]],
                                        ab.at[p3, dsl], sem_g))
            cps.append(pltpu.async_copy(x4_hbm.at[dstb.at[p4, k]],
                                        bb.at[p3, dsl], sem_g))
        return cps

    def issue_gadd(c):
        p3 = c % 3
        p4 = c % 4
        return [pltpu.async_copy(x3_hbm.at[srcb.at[p4, k]],
                                 bb.at[p3, pl.ds(k * IR, IR)], sem_g,
                                 add=True)
                for k in range(KSUB)]

    d_iw = {c: issue_idx_w(c) for c in range(min(3, NCHUNK))}
    d_iw[0][0].wait()
    d_iw[0][1].wait()
    d_g1 = {0: issue_g1(0)}
    if NCHUNK > 1:
        d_iw[1][0].wait()
        d_iw[1][1].wait()
        d_g1[1] = issue_g1(1)
    for cp in d_g1.pop(0):
        cp.wait()
    d_ga = {0: issue_gadd(0)}
    for cp in d_ga.pop(0):
        cp.wait()
    d_iw[0][2].wait()

    d_sc = {}
    d_out = {}
    for c in range(NCHUNK):
        p3 = c % 3
        p4 = c % 4
        eo = wid * EPW + c * CH
        if c + 1 < NCHUNK:
            for cp in d_g1.pop(c + 1):
                cp.wait()
            d_ga[c + 1] = issue_gadd(c + 1)
        if c - 1 >= 0:
            d_out.pop(c - 1).wait()
        if c + 2 < NCHUNK:
            d_iw[c + 2][0].wait()
            d_iw[c + 2][1].wait()
            d_g1[c + 2] = issue_g1(c + 2)

        def mrow(e, carry2):
            lo = pl.ds(0, 16)
            hi = pl.ds(16, 16)
            pb[c % 2, e, lo] = sb[p3, e, lo] * ab[p3, e, lo]
            pb[c % 2, e, hi] = sb[p3, e, hi] * ab[p3, e, hi]
            return carry2

        lax.fori_loop(0, CH, mrow, 0, unroll=2)

        if c - 1 >= 0:
            for cp in d_sc.pop(c - 1):
                cp.wait()
        if c + 3 < NCHUNK:
            d_iw[c + 3] = issue_idx_w(c + 3)
        d_sc[c] = [pltpu.async_copy(pb.at[c % 2, pl.ds(k * IR, IR)],
                                    acc.at[srcb.at[p4, k]], sem_s, add=True)
                   for k in range(KSUB)]
        d_out[c] = pltpu.async_copy(bb.at[p3], gsum.at[pl.ds(eo, CH)],
                                    sem_o)
        if c + 1 < NCHUNK:
            for cp in d_ga.pop(c + 1):
                cp.wait()
            d_iw.pop(c + 1)[2].wait()

    for cp in d_sc.pop(NCHUNK - 1):
        cp.wait()
    d_out.pop(NCHUNK - 1).wait()

    plsc.subcore_barrier()
    pltpu.sync_copy(acc.at[pl.ds(sid * RPT, RPT)],
                    accout.at[cid, pl.ds(sid * RPT, RPT)])


def _sc_deg_body(src_hbm, accout, srcb, onesb, zb, acc):
    cid = lax.axis_index("c")
    sid = lax.axis_index("s")
    wid = sid * NC + cid

    zeros16 = jnp.zeros((16,), _F32)
    ones16 = jnp.ones((16,), _F32)

    def zrow(j, carry):
        zb[j, pl.ds(0, 16)] = zeros16
        zb[j, pl.ds(16, 16)] = zeros16
        return carry

    lax.fori_loop(0, RPT, zrow, 0, unroll=4)

    def orow(j, carry):
        onesb[j, pl.ds(0, 16)] = ones16
        onesb[j, pl.ds(16, 16)] = ones16
        return carry

    lax.fori_loop(0, IR, orow, 0, unroll=4)

    pltpu.sync_copy(zb, acc.at[pl.ds(sid * RPT, RPT)])
    plsc.subcore_barrier()

    def chunk(c, carry):
        ro = wid * NCHUNK + c
        pltpu.sync_copy(src_hbm.at[ro], srcb)
        for k in range(KSUB):
            pltpu.sync_copy(onesb, acc.at[srcb.at[k]], add=True)
        return carry

    lax.fori_loop(0, NCHUNK, chunk, 0)

    plsc.subcore_barrier()
    pltpu.sync_copy(acc.at[pl.ds(sid * RPT, RPT)],
                    accout.at[cid, pl.ds(sid * RPT, RPT)])


_SC_CACHE = {}


def _sc_kernels():
    if not _SC_CACHE:
        mesh = plsc.VectorSubcoreMesh(core_axis_name="c", subcore_axis_name="s",
                                      num_cores=NC, num_subcores=NS)
        _SC_CACHE['layer'] = pl.kernel(
            _sc_layer_body,
            out_type=(jax.ShapeDtypeStruct((NC, NP, U), _F32),
                      jax.ShapeDtypeStruct((E, U), _F32)),
            mesh=mesh,
            scratch_types=[
                pltpu.VMEM((4, KSUB, IR), jnp.int32),
                pltpu.VMEM((4, KSUB, IR), jnp.int32),
                pltpu.VMEM((3, CH, U), _F32),
                pltpu.VMEM((3, CH, U), _F32),
                pltpu.VMEM((3, CH, U), _F32),
                pltpu.VMEM((2, CH, U), _F32),
                pltpu.VMEM_SHARED((NP, U), _F32),
                pltpu.SemaphoreType.DMA,
                pltpu.SemaphoreType.DMA,
                pltpu.SemaphoreType.DMA,
                pltpu.SemaphoreType.DMA,
                pltpu.SemaphoreType.DMA,
            ],
            compiler_params=pltpu.CompilerParams(use_tc_tiling_on_sc=False),
            interpret=False,
        )
        _SC_CACHE['deg'] = pl.kernel(
            _sc_deg_body,
            out_type=jax.ShapeDtypeStruct((NC, NP, U), _F32),
            mesh=mesh,
            scratch_types=[
                pltpu.VMEM((KSUB, IR), jnp.int32),
                pltpu.VMEM((IR, U), _F32),
                pltpu.VMEM((RPT, U), _F32),
                pltpu.VMEM_SHARED((NP, U), _F32),
            ],
            compiler_params=pltpu.CompilerParams(use_tc_tiling_on_sc=False),
            interpret=False,
        )
    return _SC_CACHE


def _run_sc_layer(s, x2t, x3t, x4t, src2, dst2):
    return _sc_kernels()['layer'](s, x2t, x3t, x4t, src2, dst2)


def _run_sc_deg(src2):
    return _sc_kernels()['deg'](src2)


def _bd(w):
    return jnp.kron(jnp.eye(4, dtype=_F32), w.T)


def _tile128(v):
    return jnp.tile(v.reshape(1, U), (1, 4)).reshape(1, 128)


def _branch(feat, edge_attr, src2, dst2, rd, p):
    feats = feat.shape[1]
    xr = feat.reshape(NR, 4 * feats)
    m0 = jnp.kron(jnp.eye(4, dtype=_F32), p['v_lin0_W'].T)
    b0 = _tile128(p['v_lin0_b'])

    fold = jnp.tile(jnp.eye(U, dtype=_F32), (4, 1))
    foldt = fold.T

    def layer_mats(i):
        bds = [_bd(p['v1_W'][i]), _bd(p['v2_W'][i]),
               _bd(p['v3_W'][i]), _bd(p['v4_W'][i])]
        bt = jnp.stack([
            _tile128(p['v1_b'][i])[0],
            _tile128(p['v2_b'][i])[0],
            _tile128(p['v3_b'][i])[0],
            _tile128(p['v4_b'][i] + p['e0_b'][i])[0],
        ], axis=0)
        return bds, bt

    bds0, bt0 = layer_mats(0)
    x0, x1t, x2t, x3t, x4t = _call_node_init(xr, m0, b0, bds0, bt0)

    er = edge_attr.reshape(ER, 4)
    me = jnp.kron(jnp.eye(4, dtype=_F32), p['e_lin0_W'].T)
    be = _tile128(p['e_lin0_b'])
    w, s = _call_edge_init(er, me, be)

    for i in range(DEPTH):
        x2flat = x2t.reshape(N, U)
        x3flat = x3t.reshape(N, U)
        x4flat = x4t.reshape(N, U)
        sflat = s.reshape(E, U)
        accp, gsum = _run_sc_layer(sflat, x2flat, x3flat, x4flat, src2, dst2)
        g128 = gsum.reshape(ER, 128)

        bde = _bd(p['e0_W'][i])
        st0, st1 = _call_edge_stats(w, g128, bde)
        s0 = jnp.sum(st0.reshape(4, U), axis=0)
        s1 = jnp.sum(st1.reshape(4, U), axis=0)
        mu = s0 / E
        var = jnp.maximum(s1 / E - mu * mu, 0.0)
        scale = p['e_bn_g'][i] * lax.rsqrt(var + 1e-5)
        shift = p['e_bn_b'][i] - mu * scale
        sc128 = _tile128(scale)
        sh128 = _tile128(shift)

        last = (i == DEPTH - 1)
        if last:
            (w,) = _call_edge_update(w, g128, bde, sc128, sh128, True)
        else:
            w, s = _call_edge_update(w, g128, bde, sc128, sh128, False)

            p0 = accp[0, :N].reshape(NR, 128)
            p1 = accp[1, :N].reshape(NR, 128)
            gv = _tile128(p['v_bn_g'][i])
            bv = _tile128(p['v_bn_b'][i])
            bdsn, btn = layer_mats(i + 1)
            x0, x1t, x2t, x3t, x4t = _call_node_update(
                x0, x1t, p0, p1, rd, fold, foldt, gv, bv, bdsn, btn)

    return w.reshape(E, U)


def kernel(x, y, edge_index, edge_attr, params_cvrp, params_tw):
    src = edge_index[0]
    dst = edge_index[1]
    src2 = src.reshape(NW * NCHUNK, KSUB, IR)
    dst2 = dst.reshape(NW * NCHUNK, KSUB, IR)

    degp = _run_sc_deg(src2)
    deg = degp[0, :N, 0] + degp[1, :N, 0]
    rdeg = 1.0 / jnp.clip(deg, 1.0)
    rd = jnp.broadcast_to(rdeg[:, None], (N, U)).reshape(NR, 128)

    w_cvrp = _branch(x, edge_attr, src2, dst2, rd, params_cvrp)
    w_tw = _branch(y, edge_attr, src2, dst2, rd, params_tw)
    return (w_cvrp, w_tw)
